# baseline (speedup 1.0000x reference)
import dataclasses
import functools

import jax
import jax.numpy as jnp
from jax import lax
from jax.experimental import pallas as pl
from jax.experimental.pallas import tpu as pltpu
from jax.experimental.pallas import tpu_sc as plsc

NS = 16
LANES = 16
NPAD = 10240
HR, HC = 80, 128
NPARTS = 4
COVER = 2560

_mesh1 = dict(core_axis_name="c", subcore_axis_name="s", num_cores=1)


@jax.jit
def _sc_degree(dst3d):
    _, nch, ch = dst3d.shape
    mesh = plsc.VectorSubcoreMesh(**_mesh1)
    cp = dataclasses.replace(pltpu.CompilerParams(),
                             needs_layout_passes=False)

    @functools.partial(
        pl.kernel, out_type=jax.ShapeDtypeStruct((NS, HR, HC), jnp.float32),
        mesh=mesh, compiler_params=cp,
        scratch_types=[pltpu.VMEM((1, nch, ch), jnp.int32),
                       pltpu.VMEM((1, HR, HC), jnp.float32)])
    def run(dst_hbm, cnt_hbm, dstv, hist):
        s = lax.axis_index("s")

        @pl.loop(0, HR)
        def _(r):
            @pl.loop(0, HC, step=LANES)
            def _(k):
                hist.at[0, r, pl.ds(k, LANES)][...] = jnp.zeros(
                    (LANES,), jnp.float32)

        pltpu.sync_copy(dst_hbm.at[pl.ds(s, 1)], dstv)

        ones16 = jnp.ones((LANES,), jnp.float32)
        zeros16 = jnp.zeros((LANES,), jnp.int32)

        @pl.loop(0, nch)
        def _(i):
            @pl.loop(0, ch, step=LANES)
            def _(k):
                idx16 = dstv.at[0, i, pl.ds(k, LANES)][...]
                r16 = lax.shift_right_logical(idx16, 7)
                c16 = lax.bitwise_and(idx16, 127)
                plsc.addupdate_scatter(hist, [zeros16, r16, c16], ones16)

        pltpu.sync_copy(hist, cnt_hbm.at[pl.ds(s, 1)])

    return run(dst3d)


@jax.jit
def _sc_localize(dst3d):
    _, nch, ch = dst3d.shape
    mesh = plsc.VectorSubcoreMesh(**_mesh1)

    @functools.partial(
        pl.kernel,
        out_type=jax.ShapeDtypeStruct((NPARTS, NS, nch, ch), jnp.int32),
        mesh=mesh,
        scratch_types=[pltpu.VMEM((nch, ch), jnp.int32),
                       pltpu.VMEM((nch, ch), jnp.int32)])
    def run(dst_hbm, ldst_hbm, dstv, ldstv):
        s = lax.axis_index("s")
        pltpu.sync_copy(dst_hbm.at[s], dstv)

        for h in range(NPARTS):
            off = h * COVER

            @pl.loop(0, nch)
            def _(i):
                @pl.loop(0, ch, step=LANES)
                def _(k):
                    dv = dstv.at[i].at[pl.ds(k, LANES)][...]
                    lv = dv - off
                    ok = (lv >= 0) & (lv < COVER)
                    ldstv.at[i].at[pl.ds(k, LANES)][...] = jnp.where(
                        ok, lv, COVER)

            pltpu.sync_copy(ldstv, ldst_hbm.at[h, s])

    return run(dst3d)


@jax.jit
def _sc_part_segment_sum(x, src3d, ldst2):
    _, d = x.shape
    _, nch, ch = src3d.shape
    rows_sub = COVER // NS
    zr = rows_sub // 4
    arows = COVER + 8
    mesh = plsc.VectorSubcoreMesh(core_axis_name="c", subcore_axis_name="s")

    scratch = [
        pltpu.VMEM((nch, ch), jnp.int32),
        pltpu.VMEM((nch, ch), jnp.int32),
        pltpu.VMEM((ch, d), jnp.float32),
        pltpu.VMEM((ch, d), jnp.float32),
        pltpu.VMEM((zr, d), jnp.float32),
        pltpu.VMEM_SHARED((arows, d), jnp.float32),
        pltpu.SemaphoreType.DMA,
        pltpu.SemaphoreType.DMA,
    ]

    @functools.partial(
        pl.kernel, out_type=jax.ShapeDtypeStruct((2, COVER, d), jnp.float32),
        mesh=mesh, scratch_types=scratch)
    def run(x_hbm, src_hbm, ldst_hbm, out_hbm, srcv, ldstv, buf0, buf1,
            zbuf, acc, sem0, sem1):
        c = lax.axis_index("c")
        s = lax.axis_index("s")

        @pl.loop(0, zr)
        def _(i):
            @pl.loop(0, d, step=LANES)
            def _(j):
                zbuf.at[i, pl.ds(j, LANES)][...] = jnp.zeros(
                    (LANES,), jnp.float32)

        r0 = s * rows_sub
        for k in range(4):
            pltpu.sync_copy(zbuf, acc.at[pl.ds(r0 + k * zr, zr)])

        pltpu.sync_copy(src_hbm.at[s], srcv)
        pltpu.sync_copy(ldst_hbm.at[c, s], ldstv)
        plsc.subcore_barrier()

        bufs = (buf0, buf1)
        sems = (sem0, sem1)
        pltpu.async_copy(x_hbm.at[srcv.at[0]], buf0, sem0)
        pltpu.async_copy(x_hbm.at[srcv.at[1]], buf1, sem1)

        @pl.loop(0, nch, step=2)
        def _(j):
            for t in range(2):
                pltpu.make_async_copy(
                    x_hbm.at[srcv.at[j + t]], bufs[t], sems[t]).wait()
                pltpu.async_copy(
                    bufs[t], acc.at[ldstv.at[j + t]], sems[t], add=True)

            for t in range(2):
                @pl.when(j + t + 2 < nch)
                def _(t=t):
                    pltpu.make_async_copy(
                        bufs[t], acc.at[ldstv.at[j + t]], sems[t]).wait()
                    pltpu.async_copy(
                        x_hbm.at[srcv.at[j + t + 2]], bufs[t], sems[t])

        for t in range(2):
            pltpu.make_async_copy(
                bufs[t], acc.at[ldstv.at[nch - 2 + t]], sems[t]).wait()

        plsc.subcore_barrier()
        pltpu.sync_copy(acc.at[pl.ds(r0, rows_sub)],
                        out_hbm.at[c, pl.ds(r0, rows_sub)])

    return run(x, src3d, ldst2)


def _tc_sum_counts(cnts):
    def body(c_ref, o_ref):
        o_ref[...] = jnp.sum(c_ref[...], axis=0)

    return pl.pallas_call(
        body,
        out_shape=jax.ShapeDtypeStruct((HR, HC), jnp.float32),
    )(cnts)


def _combine(yp, cnt_col, x_in, w_l, b_l2, w_r, relu_flag):
    n, d = x_in.shape
    bt = 1000

    def body(yp_ref, cnt_ref, x_ref, wl_ref, bl_ref, wr_ref, fl_ref, o_ref):
        cn = cnt_ref[...]
        rc = 1.0 / jnp.maximum(cn, 1.0)
        mask = jnp.minimum(cn, 1.0)
        h = jnp.dot(yp_ref[...], wl_ref[...],
                    preferred_element_type=jnp.float32) * rc
        h = h + mask * bl_ref[...]
        h = h + jnp.dot(x_ref[...], wr_ref[...],
                        preferred_element_type=jnp.float32)
        h = jnp.where(fl_ref[0, 0] > 0.5, jnp.maximum(h, 0.0), h)
        o_ref[...] = h

    return pl.pallas_call(
        body,
        grid=(n // bt,),
        in_specs=[
            pl.BlockSpec((bt, d), lambda i: (i, 0)),
            pl.BlockSpec((bt, 1), lambda i: (i, 0)),
            pl.BlockSpec((bt, d), lambda i: (i, 0)),
            pl.BlockSpec((d, d), lambda i: (0, 0)),
            pl.BlockSpec((1, d), lambda i: (0, 0)),
            pl.BlockSpec((d, d), lambda i: (0, 0)),
            pl.BlockSpec((1, 1), lambda i: (0, 0)),
        ],
        out_specs=pl.BlockSpec((bt, d), lambda i: (i, 0)),
        out_shape=jax.ShapeDtypeStruct((n, d), jnp.float32),
    )(yp, cnt_col, x_in, w_l, b_l2, w_r, relu_flag)


def kernel(x, edges, W_l0, b_l0, W_r0, W_l1, b_l1, W_r1):
    n, d = x.shape
    e = edges.shape[1]
    ch = 80
    nch = e // (NS * ch)
    src3d = edges[0].reshape(NS, nch, ch)
    dst3d = edges[1].reshape(NS, nch, ch)

    relu_on = jnp.full((1, 1), 1.0, jnp.float32)
    relu_off = jnp.full((1, 1), 0.0, jnp.float32)

    cnts = _sc_degree(dst3d)
    cnt_col = _tc_sum_counts(cnts).reshape(NPAD, 1)
    ldst = _sc_localize(dst3d)

    def layer(xc, w_l, b_l, w_r, flag):
        y01 = _sc_part_segment_sum(xc, src3d, ldst[0:2])
        y23 = _sc_part_segment_sum(xc, src3d, ldst[2:4])
        y = jnp.concatenate([y01[0], y01[1], y23[0], y23[1]], axis=0)[:n]
        return _combine(y, cnt_col[:n], xc, w_l, b_l.reshape(1, d), w_r,
                        flag)

    out1 = layer(x, W_l0, b_l0, W_r0, relu_on)
    out = layer(out1, W_l1, b_l1, W_r1, relu_off)
    return out


# speedup vs baseline: 1.0348x; 1.0348x over previous
import dataclasses
import functools

import jax
import jax.numpy as jnp
from jax import lax
from jax.experimental import pallas as pl
from jax.experimental.pallas import tpu as pltpu
from jax.experimental.pallas import tpu_sc as plsc

NS = 16
LANES = 16
NPAD = 10240
HR, HC = 80, 128
NPARTS = 4
COVER = 2560

_mesh1 = dict(core_axis_name="c", subcore_axis_name="s", num_cores=1)


@jax.jit
def _sc_degree(dst3d):
    _, nch, ch = dst3d.shape
    mesh = plsc.VectorSubcoreMesh(**_mesh1)
    cp = dataclasses.replace(pltpu.CompilerParams(),
                             needs_layout_passes=False)

    @functools.partial(
        pl.kernel, out_type=jax.ShapeDtypeStruct((NS, HR, HC), jnp.float32),
        mesh=mesh, compiler_params=cp,
        scratch_types=[pltpu.VMEM((1, nch, ch), jnp.int32),
                       pltpu.VMEM((1, HR, HC), jnp.float32)])
    def run(dst_hbm, cnt_hbm, dstv, hist):
        s = lax.axis_index("s")

        @pl.loop(0, HR)
        def _(r):
            @pl.loop(0, HC, step=LANES)
            def _(k):
                hist.at[0, r, pl.ds(k, LANES)][...] = jnp.zeros(
                    (LANES,), jnp.float32)

        pltpu.sync_copy(dst_hbm.at[pl.ds(s, 1)], dstv)

        ones16 = jnp.ones((LANES,), jnp.float32)
        zeros16 = jnp.zeros((LANES,), jnp.int32)

        @pl.loop(0, nch)
        def _(i):
            @pl.loop(0, ch, step=LANES)
            def _(k):
                idx16 = dstv.at[0, i, pl.ds(k, LANES)][...]
                r16 = lax.shift_right_logical(idx16, 7)
                c16 = lax.bitwise_and(idx16, 127)
                plsc.addupdate_scatter(hist, [zeros16, r16, c16], ones16)

        pltpu.sync_copy(hist, cnt_hbm.at[pl.ds(s, 1)])

    return run(dst3d)


@jax.jit
def _sc_localize(dst3d):
    _, nch, ch = dst3d.shape
    mesh = plsc.VectorSubcoreMesh(**_mesh1)

    @functools.partial(
        pl.kernel,
        out_type=jax.ShapeDtypeStruct((NPARTS, NS, nch, ch), jnp.int32),
        mesh=mesh,
        scratch_types=[pltpu.VMEM((nch, ch), jnp.int32),
                       pltpu.VMEM((nch, ch), jnp.int32)])
    def run(dst_hbm, ldst_hbm, dstv, ldstv):
        s = lax.axis_index("s")
        pltpu.sync_copy(dst_hbm.at[s], dstv)

        for h in range(NPARTS):
            off = h * COVER

            @pl.loop(0, nch)
            def _(i):
                @pl.loop(0, ch, step=LANES)
                def _(k):
                    dv = dstv.at[i].at[pl.ds(k, LANES)][...]
                    lv = dv - off
                    ok = (lv >= 0) & (lv < COVER)
                    ldstv.at[i].at[pl.ds(k, LANES)][...] = jnp.where(
                        ok, lv, COVER)

            pltpu.sync_copy(ldstv, ldst_hbm.at[h, s])

    return run(dst3d)


@jax.jit
def _sc_part_segment_sum(x, src3d, ldst2):
    _, d = x.shape
    _, nch, ch = src3d.shape
    rows_sub = COVER // NS
    zr = rows_sub // 4
    arows = COVER + 8
    mesh = plsc.VectorSubcoreMesh(core_axis_name="c", subcore_axis_name="s")

    scratch = [
        pltpu.VMEM((nch, ch), jnp.int32),
        pltpu.VMEM((nch, ch), jnp.int32),
        pltpu.VMEM((ch, d), jnp.float32),
        pltpu.VMEM((ch, d), jnp.float32),
        pltpu.VMEM((zr, d), jnp.float32),
        pltpu.VMEM_SHARED((arows, d), jnp.float32),
        pltpu.SemaphoreType.DMA,
        pltpu.SemaphoreType.DMA,
    ]

    @functools.partial(
        pl.kernel, out_type=jax.ShapeDtypeStruct((2, COVER, d), jnp.float32),
        mesh=mesh, scratch_types=scratch)
    def run(x_hbm, src_hbm, ldst_hbm, out_hbm, srcv, ldstv, bufa, bufb,
            zbuf, acc, sema, semb):
        c = lax.axis_index("c")
        s = lax.axis_index("s")

        @pl.loop(0, zr)
        def _(i):
            @pl.loop(0, d, step=LANES)
            def _(j):
                zbuf.at[i, pl.ds(j, LANES)][...] = jnp.zeros(
                    (LANES,), jnp.float32)

        r0 = s * rows_sub
        for k in range(4):
            pltpu.sync_copy(zbuf, acc.at[pl.ds(r0 + k * zr, zr)])

        pltpu.sync_copy(src_hbm.at[s], srcv)
        pltpu.sync_copy(ldst_hbm.at[c, s], ldstv)
        plsc.subcore_barrier()

        pltpu.async_copy(x_hbm.at[srcv.at[0]], bufa, sema)
        pltpu.async_copy(x_hbm.at[srcv.at[1]], bufb, semb)

        @pl.loop(0, nch, step=2)
        def _(j):
            pltpu.make_async_copy(x_hbm.at[srcv.at[j]], bufa, sema).wait()
            pltpu.sync_copy(bufa, acc.at[ldstv.at[j]], add=True)

            @pl.when(j + 2 < nch)
            def _():
                pltpu.async_copy(x_hbm.at[srcv.at[j + 2]], bufa, sema)

            pltpu.make_async_copy(x_hbm.at[srcv.at[j + 1]], bufb, semb).wait()
            pltpu.sync_copy(bufb, acc.at[ldstv.at[j + 1]], add=True)

            @pl.when(j + 3 < nch)
            def _():
                pltpu.async_copy(x_hbm.at[srcv.at[j + 3]], bufb, semb)

        plsc.subcore_barrier()
        pltpu.sync_copy(acc.at[pl.ds(r0, rows_sub)],
                        out_hbm.at[c, pl.ds(r0, rows_sub)])

    return run(x, src3d, ldst2)


def _tc_sum_counts(cnts):
    def body(c_ref, o_ref):
        o_ref[...] = jnp.sum(c_ref[...], axis=0)

    return pl.pallas_call(
        body,
        out_shape=jax.ShapeDtypeStruct((HR, HC), jnp.float32),
    )(cnts)


def _combine(yp, cnt_col, x_in, w_l, b_l2, w_r, relu_flag):
    n, d = x_in.shape
    bt = 1000

    def body(yp_ref, cnt_ref, x_ref, wl_ref, bl_ref, wr_ref, fl_ref, o_ref):
        cn = cnt_ref[...]
        rc = 1.0 / jnp.maximum(cn, 1.0)
        mask = jnp.minimum(cn, 1.0)
        h = jnp.dot(yp_ref[...], wl_ref[...],
                    preferred_element_type=jnp.float32) * rc
        h = h + mask * bl_ref[...]
        h = h + jnp.dot(x_ref[...], wr_ref[...],
                        preferred_element_type=jnp.float32)
        h = jnp.where(fl_ref[0, 0] > 0.5, jnp.maximum(h, 0.0), h)
        o_ref[...] = h

    return pl.pallas_call(
        body,
        grid=(n // bt,),
        in_specs=[
            pl.BlockSpec((bt, d), lambda i: (i, 0)),
            pl.BlockSpec((bt, 1), lambda i: (i, 0)),
            pl.BlockSpec((bt, d), lambda i: (i, 0)),
            pl.BlockSpec((d, d), lambda i: (0, 0)),
            pl.BlockSpec((1, d), lambda i: (0, 0)),
            pl.BlockSpec((d, d), lambda i: (0, 0)),
            pl.BlockSpec((1, 1), lambda i: (0, 0)),
        ],
        out_specs=pl.BlockSpec((bt, d), lambda i: (i, 0)),
        out_shape=jax.ShapeDtypeStruct((n, d), jnp.float32),
    )(yp, cnt_col, x_in, w_l, b_l2, w_r, relu_flag)


def kernel(x, edges, W_l0, b_l0, W_r0, W_l1, b_l1, W_r1):
    n, d = x.shape
    e = edges.shape[1]
    ch = 80
    nch = e // (NS * ch)
    src3d = edges[0].reshape(NS, nch, ch)
    dst3d = edges[1].reshape(NS, nch, ch)

    relu_on = jnp.full((1, 1), 1.0, jnp.float32)
    relu_off = jnp.full((1, 1), 0.0, jnp.float32)

    cnts = _sc_degree(dst3d)
    cnt_col = _tc_sum_counts(cnts).reshape(NPAD, 1)
    ldst = _sc_localize(dst3d)

    def layer(xc, w_l, b_l, w_r, flag):
        y01 = _sc_part_segment_sum(xc, src3d, ldst[0:2])
        y23 = _sc_part_segment_sum(xc, src3d, ldst[2:4])
        y = jnp.concatenate([y01[0], y01[1], y23[0], y23[1]], axis=0)[:n]
        return _combine(y, cnt_col[:n], xc, w_l, b_l.reshape(1, d), w_r,
                        flag)

    out1 = layer(x, W_l0, b_l0, W_r0, relu_on)
    out = layer(out1, W_l1, b_l1, W_r1, relu_off)
    return out
